# baseline (speedup 1.0000x reference)
import functools

import jax
import jax.numpy as jnp
from jax import lax
from jax.experimental import pallas as pl
from jax.experimental.pallas import tpu as pltpu
from jax.experimental.pallas import tpu_sc as plsc

_LANES_PER_BLOCK = 16384
_SC_CHUNK = 512


def _tc_body(word_ref, emb_ref, w_ref, b_ref, out_ref):
    wblk = word_ref[0]
    lin = jax.lax.dot_general(
        w_ref[...], wblk, (((0,), (0,)), ((), ())),
        preferred_element_type=jnp.float32) + b_ref[...]
    emb = jnp.transpose(emb_ref[0][:, :32], (1, 0))
    out_ref[0] = jnp.concatenate((lin, emb), axis=0)


def _make_sc_emb(S, B, EA):
    mesh = plsc.VectorSubcoreMesh(core_axis_name="c", subcore_axis_name="s")
    info = plsc.get_sparse_core_info()
    NC, NS = info.num_cores, info.num_subcores
    NW = NC * NS
    CH = _SC_CHUNK
    per_w = (S * B) // NW

    @functools.partial(
        pl.kernel,
        out_type=jax.ShapeDtypeStruct((S * B, 128), jnp.float32),
        mesh=mesh,
        scratch_types=[
            pltpu.VMEM((CH,), jnp.int32),
            pltpu.VMEM((CH, 128), jnp.float32),
            pltpu.SemaphoreType.DMA,
        ],
    )
    def sc_emb(age_hbm, tab_hbm, out_hbm, idx_v, rows_v, sem):
        wid = lax.axis_index("s") * NC + lax.axis_index("c")

        for k in range(per_w // CH):
            base = wid * per_w + k * CH
            pltpu.sync_copy(age_hbm.at[pl.ds(base, CH)], idx_v)
            pltpu.async_copy(tab_hbm.at[idx_v], rows_v, sem).wait()
            pltpu.sync_copy(rows_v, out_hbm.at[pl.ds(base, CH)])

    return sc_emb


def kernel(word, age, age_table, W, b):
    B, S, D = word.shape
    E = W.shape[1]
    A, EA = age_table.shape

    word_t = jnp.transpose(word, (1, 2, 0))
    age_t = jnp.transpose(jnp.asarray(age, jnp.int32), (1, 0))
    b_col = b.reshape(E, 1)

    tab128 = jnp.zeros((A, 128), jnp.float32).at[:, :EA].set(age_table)
    emb = _make_sc_emb(S, B, EA)(age_t.reshape(S * B), tab128)
    emb = emb.reshape(S, B, 128)

    CL = _LANES_PER_BLOCK
    grid = (S, B // CL)
    out_t = pl.pallas_call(
        _tc_body,
        grid=grid,
        in_specs=[
            pl.BlockSpec((1, D, CL), lambda s, j: (s, 0, j)),
            pl.BlockSpec((1, CL, 128), lambda s, j: (s, j, 0)),
            pl.BlockSpec((D, E), lambda s, j: (0, 0)),
            pl.BlockSpec((E, 1), lambda s, j: (0, 0)),
        ],
        out_specs=pl.BlockSpec((1, E + EA, CL), lambda s, j: (s, 0, j)),
        out_shape=jax.ShapeDtypeStruct((S, E + EA, B), jnp.float32),
    )(word_t, emb, W, b_col)
    return jnp.transpose(out_t, (2, 0, 1))


# speedup vs baseline: 6.8404x; 6.8404x over previous
import jax
import jax.numpy as jnp
from jax.experimental import pallas as pl

_LANES_PER_BLOCK = 16384


def _fused_body(word_ref, age_ref, w_ref, b_ref, tab_ref, out_ref):
    wblk = word_ref[0]
    lin = jax.lax.dot_general(
        w_ref[...], wblk, (((0,), (0,)), ((), ())),
        preferred_element_type=jnp.float32) + b_ref[...]
    age_row = age_ref[pl.ds(pl.program_id(0), 1), :]
    A = tab_ref.shape[0]
    rows = jax.lax.broadcasted_iota(jnp.int32, (A, 1), 0)
    onehot = (rows == age_row).astype(jnp.float32)
    emb = jax.lax.dot_general(
        tab_ref[...], onehot, (((0,), (0,)), ((), ())),
        preferred_element_type=jnp.float32)
    out_ref[0] = jnp.concatenate((lin, emb), axis=0)


def kernel(word, age, age_table, W, b):
    B, S, D = word.shape
    E = W.shape[1]
    A, EA = age_table.shape

    word_t = jnp.transpose(word, (1, 2, 0))
    age_t = jnp.transpose(jnp.asarray(age, jnp.int32), (1, 0))
    b_col = b.reshape(E, 1)

    CL = _LANES_PER_BLOCK
    grid = (S, B // CL)
    out_t = pl.pallas_call(
        _fused_body,
        grid=grid,
        in_specs=[
            pl.BlockSpec((1, D, CL), lambda s, j: (s, 0, j)),
            pl.BlockSpec((S, CL), lambda s, j: (0, j)),
            pl.BlockSpec((D, E), lambda s, j: (0, 0)),
            pl.BlockSpec((E, 1), lambda s, j: (0, 0)),
            pl.BlockSpec((A, EA), lambda s, j: (0, 0)),
        ],
        out_specs=pl.BlockSpec((1, E + EA, CL), lambda s, j: (s, 0, j)),
        out_shape=jax.ShapeDtypeStruct((S, E + EA, B), jnp.float32),
    )(word_t, age_t, W, b_col, age_table)
    return jnp.transpose(out_t, (2, 0, 1))
